# baseline (speedup 1.0000x reference)
import jax
import jax.numpy as jnp
from jax import lax
from jax.experimental import pallas as pl
from jax.experimental.pallas import tpu as pltpu, tpu_sc as plsc

_NUM_RAYS = 32768
_S = 64
_NB = _S + 1
_NOUT = 2 * _NB
_HPAD = 0.01
_EPS = 1e-5
_NW = 32
_RPW = _NUM_RAYS // _NW
_SUB = 1
_RG = 16 * _SUB
_G = _RPW // _RG

_U0 = 1.0 / (2 * _NB)
_DU = (1.0 - 1.0 / _NB) / (_NB - 1)


def _body(w_hbm, e_hbm, n_hbm, f_hbm, u_hbm, out_hbm,
          wv0, wv1, ev0, ev1, nv0, fv0, nv1, fv1, wT, ebT, cdfv, markv, rkv,
          segv, uv, orow0, orow1,
          sw0, sw1, se0, se1, snf0, snf1, so0, so1):
    wid = lax.axis_index("s") * 2 + lax.axis_index("c")
    lane = lax.iota(jnp.int32, 16)
    lane64 = lane * _S
    laneNB = lane * _NB
    lane130 = lane * _NOUT
    lane2 = lane * 2
    u0 = jnp.float32(_U0)
    invdu = jnp.float32(1.0) / jnp.float32(_DU)

    pltpu.sync_copy(u_hbm, uv)

    def start_in(g, wvb, evb, nfvb, sw, se, snf):
        base = wid * _RPW + g * _RG
        nvb, fvb = nfvb
        pltpu.async_copy(w_hbm.at[pl.ds(base, _RG)], wvb, sw)
        pltpu.async_copy(e_hbm.at[pl.ds(base, _RG)], evb, se)
        pltpu.async_copy(n_hbm.at[pl.ds(base, _RG)], nvb, snf)
        pltpu.async_copy(f_hbm.at[pl.ds(base, _RG)], fvb, snf)

    def wait_in(wvb, evb, nfvb, sw, se, snf):
        nvb, fvb = nfvb
        pltpu.make_async_copy(w_hbm.at[pl.ds(0, _RG)], wvb, sw).wait()
        pltpu.make_async_copy(e_hbm.at[pl.ds(0, _RG)], evb, se).wait()
        pltpu.make_async_copy(n_hbm.at[pl.ds(0, _RG)], nvb, snf).wait()
        pltpu.make_async_copy(f_hbm.at[pl.ds(0, _RG)], fvb, snf).wait()

    def wait_out(orowb, so):
        pltpu.make_async_copy(orowb, out_hbm.at[pl.ds(0, _RG)], so).wait()

    nfv0, nfv1 = (nv0, fv0), (nv1, fv1)
    start_in(0, wv0, ev0, nfv0, sw0, se0, snf0)

    bufs = ((wv0, ev0, nfv0, sw0, se0, snf0, orow0, so0),
            (wv1, ev1, nfv1, sw1, se1, snf1, orow1, so1))

    def outer(gg, _):
        for b in (0, 1):
            g = gg * 2 + b
            wvb, evb, nfvb, sw, se, snf, orowb, so = bufs[b]
            nwvb, nevb, nnfvb, nsw, nse, nsnf, _, _ = bufs[1 - b]
            base = wid * _RPW + g * _RG

            zi = jnp.zeros((16,), jnp.int32)
            wait_in(wvb, evb, nfvb, sw, se, snf)

            @pl.when(g < _G - 1)
            def _():
                start_in(g + 1, nwvb, nevb, nnfvb, nsw, nse, nsnf)

            @pl.when(g >= 2)
            def _():
                wait_out(orowb, so)

            for sub in range(_SUB):
                lane_s = lane + 16 * sub

                @plsc.parallel_loop(0, _S, unroll=4,
                                    carry=jnp.zeros((16,), jnp.float32))
                def ssum(k, acc):
                    kv = jnp.full((16,), k, jnp.int32)
                    v = plsc.load_gather(wvb, [lane_s, kv]) + jnp.float32(_HPAD)
                    wT[pl.ds(k * 16, 16)] = v
                    return acc + v

                padding = jnp.maximum(jnp.float32(0), jnp.float32(_EPS) - ssum)
                padstep = padding * jnp.float32(1.0 / _S)
                inv = jnp.float32(1.0) / (ssum + padding)

                cdfv[pl.ds(0, 16)] = jnp.zeros((16,), jnp.float32)
                markv[pl.ds(_S * 16, 16)] = zi

                @plsc.parallel_loop(0, _S, unroll=4,
                                    carry=jnp.zeros((16,), jnp.float32))
                def _mk(k, acc):
                    acc = acc + wT[pl.ds(k * 16, 16)] + padstep
                    c = jnp.minimum(jnp.float32(1.0), acc * inv)
                    cdfv[pl.ds((k + 1) * 16, 16)] = c
                    markv[pl.ds(k * 16, 16)] = zi
                    return acc

                @plsc.parallel_loop(0, _NB, unroll=4,
                                    carry=jnp.zeros((16,), jnp.int32))
                def rlast(k, rprev):
                    kv = jnp.full((16,), k, jnp.int32)
                    ebT[pl.ds(k * 16, 16)] = plsc.load_gather(evb, [lane_s, kv])
                    c = cdfv[pl.ds(k * 16, 16)]
                    y = (c - u0) * invdu
                    m = y.astype(jnp.int32)
                    m = m + (m.astype(jnp.float32) < y).astype(jnp.int32)
                    rkv[pl.ds(k * 16, 16)] = m
                    kvec = kv - 1
                    plsc.store_scatter(markv,
                                       [jnp.minimum(rprev, _S) * 16 + lane],
                                       kvec, mask=m > rprev)
                    return m

                plsc.store_scatter(markv, [jnp.minimum(rlast, _S) * 16 + lane],
                                   jnp.full((16,), _S, jnp.int32),
                                   mask=rlast <= _S)

                @plsc.parallel_loop(0, _NB, unroll=8,
                                    carry=jnp.zeros((16,), jnp.int32))
                def _seg(j, kmax):
                    kmax = jnp.maximum(kmax, markv[pl.ds(j * 16, 16)])
                    segv[pl.ds(j * 16, 16)] = kmax
                    return kmax

                near = plsc.load_gather(nfvb[0], [lane_s, zi])
                far = plsc.load_gather(nfvb[1], [lane_s, zi])
                span = far - near

                @plsc.parallel_loop(0, _NB, unroll=8)
                def _fuse(j):
                    s = segv[pl.ds(j * 16, 16)]
                    hi = jnp.minimum(s + 1, _S)
                    sidx = s * 16 + lane
                    hidx = hi * 16 + lane
                    c_lo = plsc.load_gather(cdfv, [sidx])
                    c_hi = plsc.load_gather(cdfv, [hidx])
                    b_lo = plsc.load_gather(ebT, [sidx])
                    b_hi = plsc.load_gather(ebT, [hidx])
                    uj = uv[pl.ds(j * 16, 16)]
                    denom = c_hi - c_lo
                    ok = denom > jnp.float32(1e-12)
                    sd = jnp.where(ok, denom, jnp.float32(1.0))
                    t = jnp.clip(jnp.where(ok, (uj - c_lo) / sd,
                                           jnp.float32(0.0)),
                                 jnp.float32(0.0), jnp.float32(1.0))
                    binv = b_lo + t * (b_hi - b_lo)
                    plsc.store_scatter(orowb, [lane_s, s + (j + 1)],
                                       near + binv * span)
                    r = rkv[pl.ds(j * 16, 16)]
                    ebj = ebT[pl.ds(j * 16, 16)]
                    plsc.store_scatter(orowb, [lane_s, r + j],
                                       near + ebj * span)

            pltpu.async_copy(orowb, out_hbm.at[pl.ds(base, _RG)], so)
        return 0

    lax.fori_loop(0, _G // 2, outer, 0)

    wait_out(orow0, so0)
    wait_out(orow1, so1)


@jax.jit
def kernel(weights, existing_bins, nears, fars):
    u = jnp.linspace(0.0, 1.0 - 1.0 / _NB, _NB, dtype=jnp.float32) \
        + jnp.float32(1.0 / (2 * _NB))
    uv = jnp.broadcast_to(u[:, None], (_NB, 16)).reshape(-1)

    mesh = plsc.VectorSubcoreMesh(core_axis_name="c", subcore_axis_name="s")
    f = pl.kernel(
        _body,
        out_type=jax.ShapeDtypeStruct((_NUM_RAYS, _NOUT), jnp.float32),
        mesh=mesh,
        compiler_params=pltpu.CompilerParams(needs_layout_passes=False),
        scratch_types=[
            pltpu.VMEM((_RG, _S), jnp.float32),
            pltpu.VMEM((_RG, _S), jnp.float32),
            pltpu.VMEM((_RG, _NB), jnp.float32),
            pltpu.VMEM((_RG, _NB), jnp.float32),
            pltpu.VMEM((_RG, 1), jnp.float32),
            pltpu.VMEM((_RG, 1), jnp.float32),
            pltpu.VMEM((_RG, 1), jnp.float32),
            pltpu.VMEM((_RG, 1), jnp.float32),
            pltpu.VMEM((_S * 16,), jnp.float32),
            pltpu.VMEM((_NB * 16,), jnp.float32),
            pltpu.VMEM((_NB * 16,), jnp.float32),
            pltpu.VMEM((_NB * 16,), jnp.int32),
            pltpu.VMEM((_NB * 16,), jnp.int32),
            pltpu.VMEM((_NB * 16,), jnp.int32),
            pltpu.VMEM((_NB * 16,), jnp.float32),
            pltpu.VMEM((_RG, _NOUT), jnp.float32),
            pltpu.VMEM((_RG, _NOUT), jnp.float32),
            pltpu.SemaphoreType.DMA,
            pltpu.SemaphoreType.DMA,
            pltpu.SemaphoreType.DMA,
            pltpu.SemaphoreType.DMA,
            pltpu.SemaphoreType.DMA,
            pltpu.SemaphoreType.DMA,
            pltpu.SemaphoreType.DMA,
            pltpu.SemaphoreType.DMA,
        ],
    )
    return f(weights.reshape(_NUM_RAYS, _S), existing_bins, nears, fars, uv)


# speedup vs baseline: 1.0265x; 1.0265x over previous
import jax
import jax.numpy as jnp
from jax import lax
from jax.experimental import pallas as pl
from jax.experimental.pallas import tpu as pltpu, tpu_sc as plsc

_NUM_RAYS = 32768
_S = 64
_NB = _S + 1
_NOUT = 2 * _NB
_HPAD = 0.01
_EPS = 1e-5
_NW = 32
_RPW = _NUM_RAYS // _NW
_SUB = 1
_RG = 16 * _SUB
_G = _RPW // _RG

_U0 = 1.0 / (2 * _NB)
_DU = (1.0 - 1.0 / _NB) / (_NB - 1)


def _body(w_hbm, e_hbm, n_hbm, f_hbm, u_hbm, out_hbm,
          wv0, wv1, ev0, ev1, nv0, fv0, nv1, fv1, wT, ebT, cdfv, markv, rkv,
          segv, uv, orow0, orow1,
          sw0, sw1, se0, se1, snf0, snf1, so0, so1):
    wid = lax.axis_index("s") * 2 + lax.axis_index("c")
    lane = lax.iota(jnp.int32, 16)
    lane64 = lane * _S
    laneNB = lane * _NB
    lane130 = lane * _NOUT
    lane2 = lane * 2
    u0 = jnp.float32(_U0)
    invdu = jnp.float32(1.0) / jnp.float32(_DU)

    pltpu.sync_copy(u_hbm, uv)

    def start_in(g, wvb, evb, nfvb, sw, se, snf):
        base = wid * _RPW + g * _RG
        nvb, fvb = nfvb
        pltpu.async_copy(w_hbm.at[pl.ds(base, _RG)], wvb, sw)
        pltpu.async_copy(e_hbm.at[pl.ds(base, _RG)], evb, se)
        pltpu.async_copy(n_hbm.at[pl.ds(base, _RG)], nvb, snf)
        pltpu.async_copy(f_hbm.at[pl.ds(base, _RG)], fvb, snf)

    def wait_in(wvb, evb, nfvb, sw, se, snf):
        nvb, fvb = nfvb
        pltpu.make_async_copy(w_hbm.at[pl.ds(0, _RG)], wvb, sw).wait()
        pltpu.make_async_copy(e_hbm.at[pl.ds(0, _RG)], evb, se).wait()
        pltpu.make_async_copy(n_hbm.at[pl.ds(0, _RG)], nvb, snf).wait()
        pltpu.make_async_copy(f_hbm.at[pl.ds(0, _RG)], fvb, snf).wait()

    def wait_out(orowb, so):
        pltpu.make_async_copy(orowb, out_hbm.at[pl.ds(0, _RG)], so).wait()

    nfv0, nfv1 = (nv0, fv0), (nv1, fv1)
    start_in(0, wv0, ev0, nfv0, sw0, se0, snf0)

    bufs = ((wv0, ev0, nfv0, sw0, se0, snf0, orow0, so0),
            (wv1, ev1, nfv1, sw1, se1, snf1, orow1, so1))

    def outer(gg, _):
        for b in (0, 1):
            g = gg * 2 + b
            wvb, evb, nfvb, sw, se, snf, orowb, so = bufs[b]
            nwvb, nevb, nnfvb, nsw, nse, nsnf, _, _ = bufs[1 - b]
            base = wid * _RPW + g * _RG

            zi = jnp.zeros((16,), jnp.int32)
            wait_in(wvb, evb, nfvb, sw, se, snf)

            @pl.when(g < _G - 1)
            def _():
                start_in(g + 1, nwvb, nevb, nnfvb, nsw, nse, nsnf)

            @pl.when(g >= 2)
            def _():
                wait_out(orowb, so)

            for sub in range(_SUB):
                lane_s = lane + 16 * sub

                @plsc.parallel_loop(0, _S, unroll=4,
                                    carry=jnp.zeros((16,), jnp.float32))
                def ssum(k, acc):
                    kv = jnp.full((16,), k, jnp.int32)
                    v = plsc.load_gather(wvb, [lane_s, kv]) + jnp.float32(_HPAD)
                    wT[pl.ds(k * 16, 16)] = v
                    return acc + v

                padding = jnp.maximum(jnp.float32(0), jnp.float32(_EPS) - ssum)
                padstep = padding * jnp.float32(1.0 / _S)
                inv = jnp.float32(1.0) / (ssum + padding)

                cdfv[pl.ds(0, 16)] = jnp.zeros((16,), jnp.float32)
                markv[pl.ds(_S * 16, 16)] = zi

                @plsc.parallel_loop(0, _S, unroll=4,
                                    carry=jnp.zeros((16,), jnp.float32))
                def _mk(k, acc):
                    acc = acc + wT[pl.ds(k * 16, 16)] + padstep
                    c = jnp.minimum(jnp.float32(1.0), acc * inv)
                    cdfv[pl.ds((k + 1) * 16, 16)] = c
                    markv[pl.ds(k * 16, 16)] = zi
                    return acc

                @plsc.parallel_loop(0, _NB, unroll=4,
                                    carry=jnp.zeros((16,), jnp.int32))
                def rlast(k, rprev):
                    kv = jnp.full((16,), k, jnp.int32)
                    ebT[pl.ds(k * 16, 16)] = plsc.load_gather(evb, [lane_s, kv])
                    c = cdfv[pl.ds(k * 16, 16)]
                    y = (c - u0) * invdu
                    m = y.astype(jnp.int32)
                    m = m + (m.astype(jnp.float32) < y).astype(jnp.int32)
                    rkv[pl.ds(k * 16, 16)] = m
                    kvec = kv - 1
                    plsc.store_scatter(markv,
                                       [jnp.minimum(rprev, _S) * 16 + lane],
                                       kvec, mask=m > rprev)
                    return m

                plsc.store_scatter(markv, [jnp.minimum(rlast, _S) * 16 + lane],
                                   jnp.full((16,), _S, jnp.int32),
                                   mask=rlast <= _S)

                @plsc.parallel_loop(0, _NB, unroll=8,
                                    carry=jnp.zeros((16,), jnp.int32))
                def _seg(j, kmax):
                    kmax = jnp.maximum(kmax, markv[pl.ds(j * 16, 16)])
                    segv[pl.ds(j * 16, 16)] = kmax
                    return kmax

                near = plsc.load_gather(nfvb[0], [lane_s, zi])
                far = plsc.load_gather(nfvb[1], [lane_s, zi])
                span = far - near

                @plsc.parallel_loop(0, _NB, unroll=5)
                def _fuse(j):
                    s = segv[pl.ds(j * 16, 16)]
                    hi = jnp.minimum(s + 1, _S)
                    sidx = s * 16 + lane
                    hidx = hi * 16 + lane
                    c_lo = plsc.load_gather(cdfv, [sidx])
                    c_hi = plsc.load_gather(cdfv, [hidx])
                    b_lo = plsc.load_gather(ebT, [sidx])
                    b_hi = plsc.load_gather(ebT, [hidx])
                    uj = uv[pl.ds(j * 16, 16)]
                    denom = c_hi - c_lo
                    ok = denom > jnp.float32(1e-12)
                    sd = jnp.where(ok, denom, jnp.float32(1.0))
                    t = jnp.clip(jnp.where(ok, (uj - c_lo) / sd,
                                           jnp.float32(0.0)),
                                 jnp.float32(0.0), jnp.float32(1.0))
                    binv = b_lo + t * (b_hi - b_lo)
                    plsc.store_scatter(orowb, [lane_s, s + (j + 1)],
                                       near + binv * span)
                    r = rkv[pl.ds(j * 16, 16)]
                    ebj = ebT[pl.ds(j * 16, 16)]
                    plsc.store_scatter(orowb, [lane_s, r + j],
                                       near + ebj * span)

            pltpu.async_copy(orowb, out_hbm.at[pl.ds(base, _RG)], so)
        return 0

    lax.fori_loop(0, _G // 2, outer, 0)

    wait_out(orow0, so0)
    wait_out(orow1, so1)


@jax.jit
def kernel(weights, existing_bins, nears, fars):
    u = jnp.linspace(0.0, 1.0 - 1.0 / _NB, _NB, dtype=jnp.float32) \
        + jnp.float32(1.0 / (2 * _NB))
    uv = jnp.broadcast_to(u[:, None], (_NB, 16)).reshape(-1)

    mesh = plsc.VectorSubcoreMesh(core_axis_name="c", subcore_axis_name="s")
    f = pl.kernel(
        _body,
        out_type=jax.ShapeDtypeStruct((_NUM_RAYS, _NOUT), jnp.float32),
        mesh=mesh,
        compiler_params=pltpu.CompilerParams(needs_layout_passes=False),
        scratch_types=[
            pltpu.VMEM((_RG, _S), jnp.float32),
            pltpu.VMEM((_RG, _S), jnp.float32),
            pltpu.VMEM((_RG, _NB), jnp.float32),
            pltpu.VMEM((_RG, _NB), jnp.float32),
            pltpu.VMEM((_RG, 1), jnp.float32),
            pltpu.VMEM((_RG, 1), jnp.float32),
            pltpu.VMEM((_RG, 1), jnp.float32),
            pltpu.VMEM((_RG, 1), jnp.float32),
            pltpu.VMEM((_S * 16,), jnp.float32),
            pltpu.VMEM((_NB * 16,), jnp.float32),
            pltpu.VMEM((_NB * 16,), jnp.float32),
            pltpu.VMEM((_NB * 16,), jnp.int32),
            pltpu.VMEM((_NB * 16,), jnp.int32),
            pltpu.VMEM((_NB * 16,), jnp.int32),
            pltpu.VMEM((_NB * 16,), jnp.float32),
            pltpu.VMEM((_RG, _NOUT), jnp.float32),
            pltpu.VMEM((_RG, _NOUT), jnp.float32),
            pltpu.SemaphoreType.DMA,
            pltpu.SemaphoreType.DMA,
            pltpu.SemaphoreType.DMA,
            pltpu.SemaphoreType.DMA,
            pltpu.SemaphoreType.DMA,
            pltpu.SemaphoreType.DMA,
            pltpu.SemaphoreType.DMA,
            pltpu.SemaphoreType.DMA,
        ],
    )
    return f(weights.reshape(_NUM_RAYS, _S), existing_bins, nears, fars, uv)


# speedup vs baseline: 1.0269x; 1.0004x over previous
import jax
import jax.numpy as jnp
from jax import lax
from jax.experimental import pallas as pl
from jax.experimental.pallas import tpu as pltpu, tpu_sc as plsc

_NUM_RAYS = 32768
_S = 64
_NB = _S + 1
_NOUT = 2 * _NB
_HPAD = 0.01
_EPS = 1e-5
_NW = 32
_RPW = _NUM_RAYS // _NW
_SUB = 1
_RG = 16 * _SUB
_G = _RPW // _RG

_U0 = 1.0 / (2 * _NB)
_DU = (1.0 - 1.0 / _NB) / (_NB - 1)


def _body(w_hbm, e_hbm, n_hbm, f_hbm, u_hbm, out_hbm,
          wv0, wv1, ev0, ev1, nv0, fv0, nv1, fv1, wT, ebT, cdfv, markv, rkv,
          segv, uv, orow0, orow1,
          sw0, sw1, se0, se1, snf0, snf1, so0, so1):
    wid = lax.axis_index("s") * 2 + lax.axis_index("c")
    lane = lax.iota(jnp.int32, 16)
    u0 = jnp.float32(_U0)
    invdu = jnp.float32(1.0) / jnp.float32(_DU)

    pltpu.sync_copy(u_hbm, uv)

    def start_in(g, wvb, evb, nfvb, sw, se, snf):
        base = wid * _RPW + g * _RG
        nvb, fvb = nfvb
        pltpu.async_copy(w_hbm.at[pl.ds(base, _RG)], wvb, sw)
        pltpu.async_copy(e_hbm.at[pl.ds(base, _RG)], evb, se)
        pltpu.async_copy(n_hbm.at[pl.ds(base, _RG)], nvb, snf)
        pltpu.async_copy(f_hbm.at[pl.ds(base, _RG)], fvb, snf)

    def wait_in(wvb, evb, nfvb, sw, se, snf):
        nvb, fvb = nfvb
        pltpu.make_async_copy(w_hbm.at[pl.ds(0, _RG)], wvb, sw).wait()
        pltpu.make_async_copy(e_hbm.at[pl.ds(0, _RG)], evb, se).wait()
        pltpu.make_async_copy(n_hbm.at[pl.ds(0, _RG)], nvb, snf).wait()
        pltpu.make_async_copy(f_hbm.at[pl.ds(0, _RG)], fvb, snf).wait()

    def wait_out(orowb, so):
        pltpu.make_async_copy(orowb, out_hbm.at[pl.ds(0, _RG)], so).wait()

    nfv0, nfv1 = (nv0, fv0), (nv1, fv1)
    start_in(0, wv0, ev0, nfv0, sw0, se0, snf0)

    bufs = ((wv0, ev0, nfv0, sw0, se0, snf0, orow0, so0),
            (wv1, ev1, nfv1, sw1, se1, snf1, orow1, so1))

    def outer(gg, _):
        for b in (0, 1):
            g = gg * 2 + b
            wvb, evb, nfvb, sw, se, snf, orowb, so = bufs[b]
            nwvb, nevb, nnfvb, nsw, nse, nsnf, _, _ = bufs[1 - b]
            base = wid * _RPW + g * _RG

            zi = jnp.zeros((16,), jnp.int32)
            wait_in(wvb, evb, nfvb, sw, se, snf)

            @pl.when(g < _G - 1)
            def _():
                start_in(g + 1, nwvb, nevb, nnfvb, nsw, nse, nsnf)

            @pl.when(g >= 2)
            def _():
                wait_out(orowb, so)

            for sub in range(_SUB):
                lane_s = lane + 16 * sub

                @plsc.parallel_loop(0, _S, unroll=4,
                                    carry=jnp.zeros((16,), jnp.float32))
                def ssum(k, acc):
                    kv = jnp.full((16,), k, jnp.int32)
                    v = plsc.load_gather(wvb, [lane_s, kv]) + jnp.float32(_HPAD)
                    wT[pl.ds(k * 16, 16)] = v
                    return acc + v

                padding = jnp.maximum(jnp.float32(0), jnp.float32(_EPS) - ssum)
                padstep = padding * jnp.float32(1.0 / _S)
                inv = jnp.float32(1.0) / (ssum + padding)

                cdfv[pl.ds(0, 16)] = jnp.zeros((16,), jnp.float32)
                markv[pl.ds(_S * 16, 16)] = zi

                @plsc.parallel_loop(0, _S, unroll=4,
                                    carry=jnp.zeros((16,), jnp.float32))
                def _mk(k, acc):
                    acc = acc + wT[pl.ds(k * 16, 16)] + padstep
                    c = jnp.minimum(jnp.float32(1.0), acc * inv)
                    cdfv[pl.ds((k + 1) * 16, 16)] = c
                    markv[pl.ds(k * 16, 16)] = zi
                    return acc

                @plsc.parallel_loop(0, _NB, unroll=4,
                                    carry=jnp.zeros((16,), jnp.int32))
                def rlast(k, rprev):
                    kv = jnp.full((16,), k, jnp.int32)
                    ebT[pl.ds(k * 16, 16)] = plsc.load_gather(evb, [lane_s, kv])
                    c = cdfv[pl.ds(k * 16, 16)]
                    y = (c - u0) * invdu
                    m = y.astype(jnp.int32)
                    m = m + (m.astype(jnp.float32) < y).astype(jnp.int32)
                    rkv[pl.ds(k * 16, 16)] = m
                    kvec = kv - 1
                    plsc.store_scatter(markv,
                                       [jnp.minimum(rprev, _S) * 16 + lane],
                                       kvec, mask=m > rprev)
                    return m

                plsc.store_scatter(markv, [jnp.minimum(rlast, _S) * 16 + lane],
                                   jnp.full((16,), _S, jnp.int32),
                                   mask=rlast <= _S)

                @plsc.parallel_loop(0, _NB, unroll=8,
                                    carry=jnp.zeros((16,), jnp.int32))
                def _seg(j, kmax):
                    kmax = jnp.maximum(kmax, markv[pl.ds(j * 16, 16)])
                    segv[pl.ds(j * 16, 16)] = kmax
                    return kmax

                near = plsc.load_gather(nfvb[0], [lane_s, zi])
                far = plsc.load_gather(nfvb[1], [lane_s, zi])
                span = far - near

                @plsc.parallel_loop(0, _NB, unroll=5)
                def _fuse(j):
                    s = segv[pl.ds(j * 16, 16)]
                    hi = jnp.minimum(s + 1, _S)
                    sidx = s * 16 + lane
                    hidx = hi * 16 + lane
                    c_lo = plsc.load_gather(cdfv, [sidx])
                    c_hi = plsc.load_gather(cdfv, [hidx])
                    b_lo = plsc.load_gather(ebT, [sidx])
                    b_hi = plsc.load_gather(ebT, [hidx])
                    uj = uv[pl.ds(j * 16, 16)]
                    denom = c_hi - c_lo
                    ok = denom > jnp.float32(1e-12)
                    sd = jnp.where(ok, denom, jnp.float32(1.0))
                    t = jnp.clip(jnp.where(ok, (uj - c_lo) / sd,
                                           jnp.float32(0.0)),
                                 jnp.float32(0.0), jnp.float32(1.0))
                    binv = b_lo + t * (b_hi - b_lo)
                    plsc.store_scatter(orowb, [lane_s, s + (j + 1)],
                                       near + binv * span)
                    r = rkv[pl.ds(j * 16, 16)]
                    ebj = ebT[pl.ds(j * 16, 16)]
                    plsc.store_scatter(orowb, [lane_s, r + j],
                                       near + ebj * span)

            pltpu.async_copy(orowb, out_hbm.at[pl.ds(base, _RG)], so)
        return 0

    lax.fori_loop(0, _G // 2, outer, 0)

    wait_out(orow0, so0)
    wait_out(orow1, so1)


@jax.jit
def kernel(weights, existing_bins, nears, fars):
    u = jnp.linspace(0.0, 1.0 - 1.0 / _NB, _NB, dtype=jnp.float32) \
        + jnp.float32(1.0 / (2 * _NB))
    uv = jnp.broadcast_to(u[:, None], (_NB, 16)).reshape(-1)

    mesh = plsc.VectorSubcoreMesh(core_axis_name="c", subcore_axis_name="s")
    f = pl.kernel(
        _body,
        out_type=jax.ShapeDtypeStruct((_NUM_RAYS, _NOUT), jnp.float32),
        mesh=mesh,
        compiler_params=pltpu.CompilerParams(needs_layout_passes=False),
        scratch_types=[
            pltpu.VMEM((_RG, _S), jnp.float32),
            pltpu.VMEM((_RG, _S), jnp.float32),
            pltpu.VMEM((_RG, _NB), jnp.float32),
            pltpu.VMEM((_RG, _NB), jnp.float32),
            pltpu.VMEM((_RG, 1), jnp.float32),
            pltpu.VMEM((_RG, 1), jnp.float32),
            pltpu.VMEM((_RG, 1), jnp.float32),
            pltpu.VMEM((_RG, 1), jnp.float32),
            pltpu.VMEM((_S * 16,), jnp.float32),
            pltpu.VMEM((_NB * 16,), jnp.float32),
            pltpu.VMEM((_NB * 16,), jnp.float32),
            pltpu.VMEM((_NB * 16,), jnp.int32),
            pltpu.VMEM((_NB * 16,), jnp.int32),
            pltpu.VMEM((_NB * 16,), jnp.int32),
            pltpu.VMEM((_NB * 16,), jnp.float32),
            pltpu.VMEM((_RG, _NOUT), jnp.float32),
            pltpu.VMEM((_RG, _NOUT), jnp.float32),
            pltpu.SemaphoreType.DMA,
            pltpu.SemaphoreType.DMA,
            pltpu.SemaphoreType.DMA,
            pltpu.SemaphoreType.DMA,
            pltpu.SemaphoreType.DMA,
            pltpu.SemaphoreType.DMA,
            pltpu.SemaphoreType.DMA,
            pltpu.SemaphoreType.DMA,
        ],
    )
    return f(weights.reshape(_NUM_RAYS, _S), existing_bins, nears, fars, uv)
